# baseline (speedup 1.0000x reference)
import functools
import math

import numpy as np
import jax
import jax.numpy as jnp
from jax import lax
from jax.experimental import pallas as pl
from jax.experimental.pallas import tpu as pltpu
from jax.experimental.pallas import tpu_sc as plsc

_EMD = 64
_SEQ = 200
_CHUNK = 128
_NUM_WORKERS = 32


def _pe_ext_np(seq_len: int, emd: int, chunk: int) -> np.ndarray:
    position_idx = np.arange(0, seq_len, dtype=np.float32)[:, None]
    fill = position_idx * np.exp(
        -np.arange(0, emd, 2, dtype=np.float32) / emd * math.log(10000.0)
    )
    pe = np.zeros((seq_len, emd), dtype=np.float32)
    pe[:, 0::2] = np.sin(fill)
    pe[:, 1::2] = np.cos(fill)
    return np.concatenate([pe, pe[:chunk]], axis=0)


_PE_EXT = _pe_ext_np(_SEQ, _EMD, _CHUNK)


def _build_sc_call(num_chunks: int, vocab: int):
    chunks_per_worker = num_chunks // _NUM_WORKERS
    n_rows = num_chunks * _CHUNK
    pe_rows = _SEQ + _CHUNK

    mesh = plsc.VectorSubcoreMesh(
        core_axis_name="c", subcore_axis_name="s", num_cores=2, num_subcores=16
    )

    @functools.partial(
        pl.kernel,
        out_type=jax.ShapeDtypeStruct((n_rows, _EMD), jnp.float32),
        mesh=mesh,
        compiler_params=pltpu.CompilerParams(use_tc_tiling_on_sc=False),
        scratch_types=[
            pltpu.VMEM((chunks_per_worker, _CHUNK), jnp.int32),
            pltpu.VMEM((pe_rows, _EMD), jnp.float32),
            pltpu.VMEM((_CHUNK, _EMD), jnp.float32),
            pltpu.SemaphoreType.DMA,
        ],
    )
    def sc_call(x_hbm, table_hbm, pe_hbm, out_hbm, idx_v, pe_v, rows_v, gsem):
        wid = lax.axis_index("s") * 2 + lax.axis_index("c")
        base_chunk = wid * chunks_per_worker

        pltpu.sync_copy(x_hbm.at[pl.ds(base_chunk, chunks_per_worker)], idx_v)
        pltpu.sync_copy(pe_hbm, pe_v)

        def one_chunk(g, carry):
            c = base_chunk + g
            pltpu.async_copy(table_hbm.at[idx_v.at[g]], rows_v, gsem).wait()
            start = (c * _CHUNK) % _SEQ

            def add_row(r, carry2):
                for p in range(4):
                    sl = pl.ds(p * 16, 16)
                    rows_v[r, sl] = rows_v[r, sl] + pe_v[start + r, sl]
                return carry2

            lax.fori_loop(0, _CHUNK, add_row, 0, unroll=2)
            pltpu.sync_copy(rows_v, out_hbm.at[pl.ds(c * _CHUNK, _CHUNK)])
            return carry

        lax.fori_loop(0, chunks_per_worker, one_chunk, 0)

    return sc_call


def kernel(x, table):
    b, s = x.shape
    vocab, emd = table.shape
    assert emd == _EMD and s == _SEQ
    n = b * s
    assert n % (_NUM_WORKERS * _CHUNK) == 0
    num_chunks = n // _CHUNK

    x_flat = x.reshape(num_chunks, _CHUNK).astype(jnp.int32)
    pe_ext = jnp.asarray(_PE_EXT)
    out = _build_sc_call(num_chunks, vocab)(x_flat, table, pe_ext)
    return out.reshape(b, s, emd)
